# baseline (speedup 1.0000x reference)
import functools

import jax
import jax.numpy as jnp
from jax import lax
from jax.experimental import pallas as pl
from jax.experimental.pallas import tpu as pltpu
from jax.experimental.pallas import tpu_sc as plsc

TOPK = 8
NEXP = 64
NC, NS = 2, 16
NW = NC * NS


def _matmul_block(x_ref, w_ref, logits_ref):
    xb = x_ref[...]
    wb = w_ref[...]
    logits_ref[...] = jax.lax.dot_general(
        wb, xb, (((1,), (1,)), ((), ())), preferred_element_type=jnp.float32
    )


def _tc_logits_chunk(N, D, chunk, c0):
    R = 1024

    def call(xf, W):
        return pl.pallas_call(
            _matmul_block,
            grid=(chunk // R,),
            in_specs=[
                pl.BlockSpec((R, D), lambda i: (i + c0 // R, 0)),
                pl.BlockSpec((NEXP, D), lambda i: (0, 0)),
            ],
            out_specs=pl.BlockSpec((NEXP, R), lambda i: (0, i)),
            out_shape=jax.ShapeDtypeStruct((NEXP, chunk), jnp.float32),
        )(xf, W)

    return call


def _make_sc_topk(M):
    rpw = M // NW
    mesh = plsc.VectorSubcoreMesh(core_axis_name="c", subcore_axis_name="s")

    @functools.partial(
        pl.kernel,
        out_type=[
            jax.ShapeDtypeStruct((TOPK, M), jnp.float32),
            jax.ShapeDtypeStruct((TOPK, M), jnp.int32),
        ],
        mesh=mesh,
        scratch_types=[
            pltpu.VMEM((NEXP, rpw), jnp.float32),
            pltpu.VMEM((TOPK, rpw), jnp.float32),
            pltpu.VMEM((TOPK, rpw), jnp.int32),
        ],
        compiler_params=pltpu.CompilerParams(skip_device_barrier=True),
    )
    def sc_topk(logits_hbm, probs_hbm, idx_hbm, buf, pv, iv):
        cid = lax.axis_index("c")
        sid = lax.axis_index("s")
        wid = sid * NC + cid
        base = wid * rpw
        pltpu.sync_copy(logits_hbm.at[:, pl.ds(base, rpw)], buf)

        def group(g, carry):
            r0 = g * 16
            tv = [jnp.full((16,), -jnp.inf, jnp.float32) for _ in range(TOPK)]
            ti = [jnp.zeros((16,), jnp.int32) for _ in range(TOPK)]
            for e in range(NEXP):
                v = buf[e, pl.ds(r0, 16)]
                vi = jnp.full((16,), e, jnp.int32)
                for i in range(TOPK):
                    gt = v > tv[i]
                    ntv = jnp.where(gt, v, tv[i])
                    nv = jnp.where(gt, tv[i], v)
                    nti = jnp.where(gt, vi, ti[i])
                    nvi = jnp.where(gt, ti[i], vi)
                    tv[i], ti[i], v, vi = ntv, nti, nv, nvi
            ev = [jnp.exp(t - tv[0]) for t in tv]
            tsum = ev[0]
            for k in range(1, TOPK):
                tsum = tsum + ev[k]
            rr = 1.0 / tsum
            for k in range(TOPK):
                pv[k, pl.ds(r0, 16)] = ev[k] * rr
                iv[k, pl.ds(r0, 16)] = ti[k]
            return carry

        lax.fori_loop(0, rpw // 16, group, 0)
        pltpu.sync_copy(pv, probs_hbm.at[:, pl.ds(base, rpw)])
        pltpu.sync_copy(iv, idx_hbm.at[:, pl.ds(base, rpw)])

    return sc_topk


@functools.partial(jax.jit, static_argnames=())
def kernel(x, W):
    B, T, D = x.shape
    N = B * T
    x_flat = x.reshape(N, D)
    C = 4
    chunk = N // C
    sc_topk = _make_sc_topk(chunk)
    lts = [
        _tc_logits_chunk(N, D, chunk, c * chunk)(x_flat, W) for c in range(C)
    ]
    pts, its = [], []
    for lt in lts:
        pt, it = sc_topk(lt)
        pts.append(pt)
        its.append(it)
    probs_t = jnp.concatenate(pts, axis=1)
    idx_t = jnp.concatenate(its, axis=1)
    aux_loss = jnp.array(0.0, dtype=jnp.float32)
    return (probs_t.T, idx_t.T, aux_loss)


# speedup vs baseline: 1.1073x; 1.1073x over previous
import functools

import jax
import jax.numpy as jnp
from jax.experimental import pallas as pl
from jax.experimental.pallas import tpu as pltpu

TOPK = 8
NEXP = 64
R = 1024
NBUF = 3


def _router_block(x_ref, probs_ref, idx_ref, w_ref):
    xb = x_ref[...]
    wb = w_ref[...]
    logits = jax.lax.dot_general(
        wb, xb, (((1,), (1,)), ((), ())), preferred_element_type=jnp.float32
    )

    m = jnp.max(logits, axis=0, keepdims=True)
    e = jnp.exp(logits - m)
    s = jnp.sum(e, axis=0, keepdims=True)
    p = e / s

    sub = jax.lax.broadcasted_iota(jnp.int32, p.shape, 0)
    vals = p
    top_v = []
    top_i = []
    for _ in range(TOPK):
        mv = jnp.max(vals, axis=0, keepdims=True)
        mi = jnp.min(jnp.where(vals == mv, sub, NEXP), axis=0, keepdims=True)
        top_v.append(mv)
        top_i.append(mi)
        vals = jnp.where(sub == mi, -1.0, vals)

    tv = jnp.concatenate(top_v, axis=0)
    ti = jnp.concatenate(top_i, axis=0)
    tv = tv / jnp.sum(tv, axis=0, keepdims=True)
    probs_ref[...] = tv.T
    idx_ref[...] = ti.T


def _outer(x_hbm, w_vmem, probs_hbm, idx_hbm):
    N, D = x_hbm.shape
    pipeline = pltpu.emit_pipeline(
        functools.partial(_router_block, w_ref=w_vmem),
        grid=(N // R,),
        in_specs=[
            pl.BlockSpec((R, D), lambda i: (i, 0),
                         pipeline_mode=pl.Buffered(buffer_count=NBUF)),
        ],
        out_specs=[
            pl.BlockSpec((R, TOPK), lambda i: (i, 0)),
            pl.BlockSpec((R, TOPK), lambda i: (i, 0)),
        ],
    )
    pipeline(x_hbm, probs_hbm, idx_hbm)


@functools.partial(jax.jit, static_argnames=())
def kernel(x, W):
    B, T, D = x.shape
    N = B * T
    x_flat = x.reshape(N, D)
    probs, idx = pl.pallas_call(
        _outer,
        in_specs=[
            pl.BlockSpec(memory_space=pl.ANY),
            pl.BlockSpec(memory_space=pltpu.VMEM),
        ],
        out_specs=[
            pl.BlockSpec(memory_space=pl.ANY),
            pl.BlockSpec(memory_space=pl.ANY),
        ],
        out_shape=[
            jax.ShapeDtypeStruct((N, TOPK), jnp.float32),
            jax.ShapeDtypeStruct((N, TOPK), jnp.int32),
        ],
    )(x_flat, W)
    aux_loss = jnp.array(0.0, dtype=jnp.float32)
    return (probs, idx, aux_loss)
